# Initial kernel scaffold; baseline (speedup 1.0000x reference)
import jax
import jax.numpy as jnp
from jax.experimental import pallas as pl


def kernel(x, edge_index, W, b):
    raise NotImplementedError("write your pallas kernel here")


# speedup vs baseline: 30.8051x; 30.8051x over previous
import functools

import jax
import jax.numpy as jnp
from jax import lax
from jax.experimental import pallas as pl
from jax.experimental.pallas import tpu as pltpu
from jax.experimental.pallas import tpu_sc as plsc

N = 10000
E = 320000
D = 128

NC = 2
NS = 16
NW = NC * NS
CHUNK = 128
CPT = 79
EPT = CHUNK * CPT
E_PAD = EPT * NW
NPAD = 10240
RPT = NPAD // NS
DEG_W = 16


def _sc_mesh():
    return plsc.VectorSubcoreMesh(core_axis_name="c", subcore_axis_name="s")


def _deg_body(col_hbm, out_hbm, colv, hist):
    c = lax.axis_index("c")
    s = lax.axis_index("s")
    wid = s * NC + c
    pltpu.sync_copy(col_hbm.at[wid], colv)

    def zero_body(i, _):
        hist[pl.ds(i * 16, 16)] = jnp.zeros((16,), jnp.float32)
        return 0

    lax.fori_loop(0, NPAD // 16, zero_body, 0)

    def add_body(i, _):
        idx = colv[pl.ds(i * 16, 16)]
        plsc.addupdate_scatter(hist, [idx], jnp.ones((16,), jnp.float32))
        return 0

    lax.fori_loop(0, EPT // 16, add_body, 0)
    pltpu.sync_copy(hist, out_hbm.at[wid])


_deg_kernel = functools.partial(
    pl.kernel,
    out_type=jax.ShapeDtypeStruct((NW, NPAD), jnp.float32),
    mesh=_sc_mesh(),
    scratch_types=[
        pltpu.VMEM((EPT,), jnp.int32),
        pltpu.VMEM((NPAD,), jnp.float32),
    ],
    compiler_params=pltpu.CompilerParams(needs_layout_passes=False),
)


def _agg_body(row_hbm, col_hbm, y_hbm, zeros_hbm, out_hbm,
              rowv, colv, buf, acc, sem):
    c = lax.axis_index("c")
    s = lax.axis_index("s")
    wid = s * NC + c
    pltpu.sync_copy(row_hbm.at[wid], rowv)
    pltpu.sync_copy(col_hbm.at[wid], colv)

    @pl.when(c == 0)
    def _():
        pltpu.sync_copy(y_hbm.at[pl.ds(s * RPT, RPT)],
                        acc.at[pl.ds(s * RPT, RPT)])

    @pl.when(c != 0)
    def _():
        pltpu.sync_copy(zeros_hbm, acc.at[pl.ds(s * RPT, RPT)])

    plsc.subcore_barrier()
    for j in range(CPT):
        pltpu.async_copy(y_hbm.at[rowv.at[j]], buf, sem).wait()
        pltpu.sync_copy(buf, acc.at[colv.at[j]], add=True)
    plsc.subcore_barrier()
    pltpu.sync_copy(acc.at[pl.ds(s * RPT, RPT)],
                    out_hbm.at[c, pl.ds(s * RPT, RPT)])


_agg_kernel = functools.partial(
    pl.kernel,
    out_type=jax.ShapeDtypeStruct((NC, NPAD, D), jnp.float32),
    mesh=_sc_mesh(),
    scratch_types=[
        pltpu.VMEM((CPT, CHUNK), jnp.int32),
        pltpu.VMEM((CPT, CHUNK), jnp.int32),
        pltpu.VMEM((CHUNK, D), jnp.float32),
        pltpu.VMEM_SHARED((NPAD, D), jnp.float32),
        pltpu.SemaphoreType.DMA,
    ],
)


def _matmul_body(x_ref, w_ref, dp_ref, y_ref, dinv_ref):
    acc = lax.dot_general(x_ref[...], w_ref[...],
                          (((1,), (1,)), ((), ())),
                          preferred_element_type=jnp.float32)
    deg = jnp.sum(dp_ref[...], axis=0) + 1.0
    dinv = lax.rsqrt(deg)
    y_ref[...] = acc * dinv[:, None]
    dinv_ref[...] = dinv


def _matmul(x_pad, w, dp):
    blk = 1024
    grid = NPAD // blk
    return pl.pallas_call(
        _matmul_body,
        grid=(grid,),
        in_specs=[
            pl.BlockSpec((blk, D), lambda i: (i, 0)),
            pl.BlockSpec((D, D), lambda i: (0, 0)),
            pl.BlockSpec((NW, blk), lambda i: (0, i)),
        ],
        out_specs=[
            pl.BlockSpec((blk, D), lambda i: (i, 0)),
            pl.BlockSpec((blk,), lambda i: (i,)),
        ],
        out_shape=[
            jax.ShapeDtypeStruct((NPAD, D), jnp.float32),
            jax.ShapeDtypeStruct((NPAD,), jnp.float32),
        ],
    )(x_pad, w, dp)


def _final_body(a0_ref, a1_ref, dinv_ref, b_ref, out_ref):
    out_ref[...] = ((a0_ref[...] + a1_ref[...]) * dinv_ref[...][:, None]
                    + b_ref[...])


def _final(a0, a1, dinv, b2d):
    blk = 1024
    grid = NPAD // blk
    return pl.pallas_call(
        _final_body,
        grid=(grid,),
        in_specs=[
            pl.BlockSpec((blk, D), lambda i: (i, 0)),
            pl.BlockSpec((blk, D), lambda i: (i, 0)),
            pl.BlockSpec((blk,), lambda i: (i,)),
            pl.BlockSpec((1, D), lambda i: (0, 0)),
        ],
        out_specs=pl.BlockSpec((blk, D), lambda i: (i, 0)),
        out_shape=jax.ShapeDtypeStruct((NPAD, D), jnp.float32),
    )(a0, a1, dinv, b2d)


def kernel(x, edge_index, W, b):
    row = edge_index[0]
    col = edge_index[1]

    pad_n = E_PAD - E
    pad_i = jnp.arange(pad_n, dtype=jnp.int32)
    row_p = jnp.concatenate([row, pad_i % N]).reshape(NW, CPT, CHUNK)
    col_flat = jnp.concatenate([col, N + pad_i % (NPAD - N)])
    col_p = col_flat.reshape(NW, CPT, CHUNK)

    zeros_row = jnp.zeros((RPT, D), jnp.float32)
    x_pad = jnp.zeros((NPAD, D), x.dtype).at[:N].set(x)

    dp = _deg_kernel(_deg_body)(col_flat.reshape(NW, EPT))

    y_pad, dinv = _matmul(x_pad, W, dp)

    a_part = _agg_kernel(_agg_body)(row_p, col_p, y_pad, zeros_row)

    out = _final(a_part[0], a_part[1], dinv, b.reshape(1, D))
    return out[:N]
